# baseline (speedup 1.0000x reference)
import math

import jax
import jax.numpy as jnp
from jax.experimental import pallas as pl
from jax.experimental.pallas import tpu as pltpu

_N = 2048
_D = 1024
_H = 8
_DH = 128
_BR = 256
_BQ = 1024
_CK = 512
_NB = _N // _BR
_NA = _H * (_N // _BQ)


def _nt(x, w):
    return jax.lax.dot_general(x, w, (((1,), (1,)), ((), ())),
                               preferred_element_type=jnp.float32)


def _mega_body(boxes_ref, feats_ref, w1t_ref, b1_ref, w2_ref, b2_ref,
               wi_ref, bi_ref, wo_ref, bo_ref, wl1_ref, bl1_ref,
               wl2_ref, bl2_ref, out_ref, q_s, k_s, v_s, o_s, acc_ref):
    i = pl.program_id(0)
    bf16 = jnp.bfloat16

    @pl.when(i < _NB)
    def _qkv():
        b = jnp.clip(boxes_ref[...], -10.0, 10.0)
        acc = jnp.broadcast_to(b1_ref[...], (_BR, _D))
        for c in range(4):
            acc = acc + b[:, c:c + 1] * w1t_ref[c:c + 1, :]
        t = jnp.maximum(acc, 0.0).astype(bf16)
        pos = _nt(t, w2_ref[...].astype(bf16))
        h = (feats_ref[...] + pos + b2_ref[...]).astype(bf16)
        qkv = _nt(h, wi_ref[...].astype(bf16)) + bi_ref[...]
        scale = 1.0 / math.sqrt(_DH)
        q = (qkv[:, :_D] * scale).astype(bf16)
        k = qkv[:, _D:2 * _D].astype(bf16)
        v = qkv[:, 2 * _D:].astype(bf16)
        row = i * _BR
        pat = (jax.lax.broadcasted_iota(jnp.int32, (_BR, _DH), 1) == 0)
        pat = pat.astype(bf16)
        for hh in range(_H):
            lo, hi = hh * _DH, (hh + 1) * _DH
            q_s[hh, pl.ds(row, _BR), :] = q[:, lo:hi]
            k_s[hh, pl.ds(row, _BR), :] = k[:, lo:hi]
            v_s[hh, pl.ds(row, _BR), :_DH] = v[:, lo:hi]
            v_s[hh, pl.ds(row, _BR), _DH:] = pat

    @pl.when((i >= _NB) & (i < _NB + _NA))
    def _attn():
        j = i - _NB
        nq = _N // _BQ
        hh = j // nq
        tt = j % nq
        q = q_s[hh, pl.ds(tt * _BQ, _BQ), :]
        o_acc = jnp.zeros((_BQ, 2 * _DH), jnp.float32)
        for c in range(_N // _CK):
            kc = k_s[hh, pl.ds(c * _CK, _CK), :]
            vc = v_s[hh, pl.ds(c * _CK, _CK), :]
            s = _nt(q, kc)
            e = jnp.exp(s.astype(bf16))
            o_acc = o_acc + jnp.dot(e, vc,
                                    preferred_element_type=jnp.float32)
        den = o_acc[:, _DH:_DH + 1]
        o_s[hh, pl.ds(tt * _BQ, _BQ), :] = (o_acc[:, :_DH] / den).astype(bf16)

    @pl.when(i >= _NB + _NA)
    def _post():
        r = i - (_NB + _NA)
        row = r * _BR

        @pl.when(r == 0)
        def _init():
            acc_ref[...] = jnp.zeros_like(acc_ref)

        o_t = jnp.concatenate(
            [o_s[hh, pl.ds(row, _BR), :] for hh in range(_H)], axis=1)
        h1 = _nt(o_t, wo_ref[...].astype(bf16)) + bo_ref[...]
        h2 = _nt(h1.astype(bf16), wl1_ref[...].astype(bf16)) + bl1_ref[...]
        h2 = jnp.maximum(h2, 0.0)
        acc_ref[...] += jnp.sum(h2, axis=0, keepdims=True)

        @pl.when(r == _NB - 1)
        def _fin():
            meanv = acc_ref[...] * (1.0 / _N)
            out_ref[...] = jax.lax.dot_general(
                meanv, wl2_ref[...], (((1,), (1,)), ((), ())),
                precision=jax.lax.Precision.HIGHEST,
                preferred_element_type=jnp.float32) + bl2_ref[...]


def kernel(feats, boxes, pos_w1, pos_b1, pos_w2, pos_b2,
           in_proj_w, in_proj_b, out_proj_w, out_proj_b,
           lin1_w, lin1_b, lin2_w, lin2_b):
    f32, bf16 = jnp.float32, jnp.bfloat16
    w1t = pos_w1.T
    b1 = pos_b1.reshape(1, _D)
    b2 = pos_b2.reshape(1, _D)
    bi = in_proj_b.reshape(1, 3 * _D)
    bo = out_proj_b.reshape(1, _D)
    bl1 = lin1_b.reshape(1, _D)
    bl2 = lin2_b.reshape(1, _D)

    def _rows(i):
        return (jnp.minimum(i, _NB - 1), 0)

    def _const(i):
        return (0, 0)

    out = pl.pallas_call(
        _mega_body,
        grid=(_NB + _NA + _NB,),
        in_specs=[
            pl.BlockSpec((_BR, 4), _rows),
            pl.BlockSpec((_BR, _D), _rows),
            pl.BlockSpec((4, _D), _const),
            pl.BlockSpec((1, _D), _const),
            pl.BlockSpec((_D, _D), _const),
            pl.BlockSpec((1, _D), _const),
            pl.BlockSpec((3 * _D, _D), _const),
            pl.BlockSpec((1, 3 * _D), _const),
            pl.BlockSpec((_D, _D), _const),
            pl.BlockSpec((1, _D), _const),
            pl.BlockSpec((_D, _D), _const),
            pl.BlockSpec((1, _D), _const),
            pl.BlockSpec((_D, _D), _const),
            pl.BlockSpec((1, _D), _const),
        ],
        out_specs=pl.BlockSpec((1, _D), _const),
        out_shape=jax.ShapeDtypeStruct((1, _D), f32),
        scratch_shapes=[
            pltpu.VMEM((_H, _N, _DH), bf16),
            pltpu.VMEM((_H, _N, _DH), bf16),
            pltpu.VMEM((_H, _N, 2 * _DH), bf16),
            pltpu.VMEM((_H, _N, _DH), bf16),
            pltpu.VMEM((1, _D), f32),
        ],
    )(boxes, feats, w1t, b1, pos_w2, b2, in_proj_w, bi,
      out_proj_w, bo, lin1_w, bl1, lin2_w, bl2)
    return out.reshape(_D)


# speedup vs baseline: 1.0364x; 1.0364x over previous
import math

import jax
import jax.numpy as jnp
from jax.experimental import pallas as pl
from jax.experimental.pallas import tpu as pltpu

_N = 2048
_D = 1024
_H = 8
_DH = 128
_BR = 256
_BQ = 1024
_CK = 512
_NB = _N // _BR
_NA = _H * (_N // _BQ)


def _nt(x, w):
    return jax.lax.dot_general(x, w, (((1,), (1,)), ((), ())),
                               preferred_element_type=jnp.float32)


def _mega_body(boxes_ref, feats_ref, w1t_ref, b1_ref, w2_ref, b2_ref,
               wi_ref, bi_ref, wo_ref, bo_ref, wl1_ref, bl1_ref,
               wl2_ref, bl2_ref, out_ref, q_s, k_s, v_s, o_s, acc_ref):
    i = pl.program_id(0)
    bf16 = jnp.bfloat16

    @pl.when(i < _NB)
    def _qkv():
        b = jnp.clip(boxes_ref[...], -10.0, 10.0)
        acc = jnp.broadcast_to(b1_ref[...], (_BR, _D))
        for c in range(4):
            acc = acc + b[:, c:c + 1] * w1t_ref[c:c + 1, :]
        t = jnp.maximum(acc, 0.0)
        pos = _nt(t, w2_ref[...])
        h = feats_ref[...] + pos + b2_ref[...]
        qkv = _nt(h, wi_ref[...]) + bi_ref[...]
        scale = 1.0 / math.sqrt(_DH)
        q = (qkv[:, :_D] * scale).astype(bf16)
        k = qkv[:, _D:2 * _D].astype(bf16)
        v = qkv[:, 2 * _D:].astype(bf16)
        row = i * _BR
        pat = (jax.lax.broadcasted_iota(jnp.int32, (_BR, _DH), 1) == 0)
        pat = pat.astype(bf16)
        for hh in range(_H):
            lo, hi = hh * _DH, (hh + 1) * _DH
            q_s[hh, pl.ds(row, _BR), :] = q[:, lo:hi]
            k_s[hh, pl.ds(row, _BR), :] = k[:, lo:hi]
            v_s[hh, pl.ds(row, _BR), :_DH] = v[:, lo:hi]
            v_s[hh, pl.ds(row, _BR), _DH:] = pat

    @pl.when((i >= _NB) & (i < _NB + _NA))
    def _attn():
        j = i - _NB
        nq = _N // _BQ
        hh = j // nq
        tt = j % nq
        q = q_s[hh, pl.ds(tt * _BQ, _BQ), :]
        o_acc = jnp.zeros((_BQ, 2 * _DH), jnp.float32)
        for c in range(_N // _CK):
            kc = k_s[hh, pl.ds(c * _CK, _CK), :]
            vc = v_s[hh, pl.ds(c * _CK, _CK), :]
            s = _nt(q, kc)
            e = jnp.exp(s.astype(bf16))
            o_acc = o_acc + jnp.dot(e, vc,
                                    preferred_element_type=jnp.float32)
        den = o_acc[:, _DH:_DH + 1]
        o_s[hh, pl.ds(tt * _BQ, _BQ), :] = (o_acc[:, :_DH] / den).astype(bf16)

    @pl.when(i >= _NB + _NA)
    def _post():
        r = i - (_NB + _NA)
        row = r * _BR

        @pl.when(r == 0)
        def _init():
            acc_ref[...] = jnp.zeros_like(acc_ref)

        o_t = jnp.concatenate(
            [o_s[hh, pl.ds(row, _BR), :] for hh in range(_H)], axis=1)
        h1 = _nt(o_t.astype(jnp.float32), wo_ref[...]) + bo_ref[...]
        h2 = _nt(h1, wl1_ref[...]) + bl1_ref[...]
        h2 = jnp.maximum(h2, 0.0)
        acc_ref[...] += jnp.sum(h2, axis=0, keepdims=True)

        @pl.when(r == _NB - 1)
        def _fin():
            meanv = acc_ref[...] * (1.0 / _N)
            out_ref[...] = _nt(meanv, wl2_ref[...]) + bl2_ref[...]


def kernel(feats, boxes, pos_w1, pos_b1, pos_w2, pos_b2,
           in_proj_w, in_proj_b, out_proj_w, out_proj_b,
           lin1_w, lin1_b, lin2_w, lin2_b):
    f32, bf16 = jnp.float32, jnp.bfloat16
    w1t = pos_w1.T
    b1 = pos_b1.reshape(1, _D)
    b2 = pos_b2.reshape(1, _D)
    bi = in_proj_b.reshape(1, 3 * _D)
    bo = out_proj_b.reshape(1, _D)
    bl1 = lin1_b.reshape(1, _D)
    bl2 = lin2_b.reshape(1, _D)

    def _rows(i):
        return (jnp.minimum(i, _NB - 1), 0)

    def _const(i):
        return (0, 0)

    out = pl.pallas_call(
        _mega_body,
        grid=(_NB + _NA + _NB,),
        in_specs=[
            pl.BlockSpec((_BR, 4), _rows),
            pl.BlockSpec((_BR, _D), _rows),
            pl.BlockSpec((4, _D), _const),
            pl.BlockSpec((1, _D), _const),
            pl.BlockSpec((_D, _D), _const),
            pl.BlockSpec((1, _D), _const),
            pl.BlockSpec((3 * _D, _D), _const),
            pl.BlockSpec((1, 3 * _D), _const),
            pl.BlockSpec((_D, _D), _const),
            pl.BlockSpec((1, _D), _const),
            pl.BlockSpec((_D, _D), _const),
            pl.BlockSpec((1, _D), _const),
            pl.BlockSpec((_D, _D), _const),
            pl.BlockSpec((1, _D), _const),
        ],
        out_specs=pl.BlockSpec((1, _D), _const),
        out_shape=jax.ShapeDtypeStruct((1, _D), f32),
        scratch_shapes=[
            pltpu.VMEM((_H, _N, _DH), bf16),
            pltpu.VMEM((_H, _N, _DH), bf16),
            pltpu.VMEM((_H, _N, 2 * _DH), bf16),
            pltpu.VMEM((_H, _N, _DH), bf16),
            pltpu.VMEM((1, _D), f32),
        ],
    )(boxes, feats, w1t, b1, pos_w2, b2, in_proj_w, bi,
      out_proj_w, bo, lin1_w, bl1, lin2_w, bl2)
    return out.reshape(_D)


# speedup vs baseline: 1.0532x; 1.0162x over previous
import math

import jax
import jax.numpy as jnp
from jax.experimental import pallas as pl
from jax.experimental.pallas import tpu as pltpu

_N = 2048
_D = 1024
_H = 8
_DH = 128
_BR = 256
_BQ = 1024
_CK = 512
_NB = _N // _BR
_NA = _H * (_N // _BQ)


def _nt(x, w):
    return jax.lax.dot_general(x, w, (((1,), (1,)), ((), ())),
                               preferred_element_type=jnp.float32)


def _mega_body(boxes_ref, feats_ref, w1t_ref, b1_ref, w2_ref, b2_ref,
               wi_ref, bi_ref, wo_ref, bo_ref, wl1_ref, bl1_ref,
               wl2_ref, bl2_ref, out_ref, q_s, k_s, v_s, o_s, acc_ref,
               f_s, fb_ref):
    i = pl.program_id(0)
    bf16 = jnp.bfloat16

    @pl.when(i < _NB)
    def _qkv():
        b = jnp.clip(boxes_ref[...], -10.0, 10.0)
        acc = jnp.broadcast_to(b1_ref[...], (_BR, _D))
        for c in range(4):
            acc = acc + b[:, c:c + 1] * w1t_ref[c:c + 1, :]
        t = jnp.maximum(acc, 0.0)
        pos = _nt(t, w2_ref[...])
        h = feats_ref[...] + pos + b2_ref[...]
        qkv = _nt(h, wi_ref[...]) + bi_ref[...]
        scale = math.log2(math.e) / math.sqrt(_DH)
        q = (qkv[:, :_D] * scale).astype(bf16)
        k = qkv[:, _D:2 * _D].astype(bf16)
        v = qkv[:, 2 * _D:].astype(bf16)
        row = i * _BR
        pat = (jax.lax.broadcasted_iota(jnp.int32, (_BR, _DH), 1) == 0)
        pat = pat.astype(bf16)
        for hh in range(_H):
            lo, hi = hh * _DH, (hh + 1) * _DH
            q_s[hh, pl.ds(row, _BR), :] = q[:, lo:hi]
            k_s[hh, pl.ds(row, _BR), :] = k[:, lo:hi]
            v_s[hh, pl.ds(row, _BR), :_DH] = v[:, lo:hi]
            v_s[hh, pl.ds(row, _BR), _DH:] = pat

    @pl.when((i >= _NB) & (i < _NB + _NA))
    def _attn():
        j = i - _NB
        nq = _N // _BQ
        hh = j // nq
        tt = j % nq
        q = q_s[hh, pl.ds(tt * _BQ, _BQ), :]
        o_acc = jnp.zeros((_BQ, 2 * _DH), jnp.float32)
        for c in range(_N // _CK):
            kc = k_s[hh, pl.ds(c * _CK, _CK), :]
            vc = v_s[hh, pl.ds(c * _CK, _CK), :]
            s = _nt(q, kc)
            e = jnp.exp2(s.astype(bf16))
            o_acc = o_acc + jnp.dot(e, vc,
                                    preferred_element_type=jnp.float32)
        den = o_acc[:, _DH:_DH + 1]
        o_s[hh, pl.ds(tt * _BQ, _BQ), :] = (o_acc[:, :_DH] / den).astype(bf16)

    @pl.when(i >= _NB + _NA)
    def _post():
        r = i - (_NB + _NA)
        row = r * _BR

        @pl.when(r == 0)
        def _init():
            acc_ref[...] = jnp.zeros_like(acc_ref)
            f = jax.lax.dot_general(wl1_ref[...], wo_ref[...],
                                    (((1,), (0,)), ((), ())),
                                    preferred_element_type=jnp.float32)
            f_s[...] = f.astype(jnp.bfloat16)
            fb_ref[...] = _nt(bo_ref[...], wl1_ref[...]) + bl1_ref[...]

        o_t = jnp.concatenate(
            [o_s[hh, pl.ds(row, _BR), :] for hh in range(_H)], axis=1)
        h2 = _nt(o_t, f_s[...]) + fb_ref[...]
        h2 = jnp.maximum(h2, 0.0)
        acc_ref[...] += jnp.sum(h2, axis=0, keepdims=True)

        @pl.when(r == _NB - 1)
        def _fin():
            meanv = acc_ref[...] * (1.0 / _N)
            out_ref[...] = _nt(meanv, wl2_ref[...]) + bl2_ref[...]


def kernel(feats, boxes, pos_w1, pos_b1, pos_w2, pos_b2,
           in_proj_w, in_proj_b, out_proj_w, out_proj_b,
           lin1_w, lin1_b, lin2_w, lin2_b):
    f32, bf16 = jnp.float32, jnp.bfloat16
    w1t = pos_w1.T
    b1 = pos_b1.reshape(1, _D)
    b2 = pos_b2.reshape(1, _D)
    bi = in_proj_b.reshape(1, 3 * _D)
    bo = out_proj_b.reshape(1, _D)
    bl1 = lin1_b.reshape(1, _D)
    bl2 = lin2_b.reshape(1, _D)

    def _rows(i):
        return (jnp.minimum(i, _NB - 1), 0)

    def _const(i):
        return (0, 0)

    out = pl.pallas_call(
        _mega_body,
        grid=(_NB + _NA + _NB,),
        in_specs=[
            pl.BlockSpec((_BR, 4), _rows),
            pl.BlockSpec((_BR, _D), _rows),
            pl.BlockSpec((4, _D), _const),
            pl.BlockSpec((1, _D), _const),
            pl.BlockSpec((_D, _D), _const),
            pl.BlockSpec((1, _D), _const),
            pl.BlockSpec((3 * _D, _D), _const),
            pl.BlockSpec((1, 3 * _D), _const),
            pl.BlockSpec((_D, _D), _const),
            pl.BlockSpec((1, _D), _const),
            pl.BlockSpec((_D, _D), _const),
            pl.BlockSpec((1, _D), _const),
            pl.BlockSpec((_D, _D), _const),
            pl.BlockSpec((1, _D), _const),
        ],
        out_specs=pl.BlockSpec((1, _D), _const),
        out_shape=jax.ShapeDtypeStruct((1, _D), f32),
        scratch_shapes=[
            pltpu.VMEM((_H, _N, _DH), bf16),
            pltpu.VMEM((_H, _N, _DH), bf16),
            pltpu.VMEM((_H, _N, 2 * _DH), bf16),
            pltpu.VMEM((_H, _N, _DH), bf16),
            pltpu.VMEM((1, _D), f32),
            pltpu.VMEM((_D, _D), bf16),
            pltpu.VMEM((1, _D), f32),
        ],
    )(boxes, feats, w1t, b1, pos_w2, b2, in_proj_w, bi,
      out_proj_w, bo, lin1_w, bl1, lin2_w, bl2)
    return out.reshape(_D)


# speedup vs baseline: 1.1331x; 1.0759x over previous
import math

import jax
import jax.numpy as jnp
from jax.experimental import pallas as pl
from jax.experimental.pallas import tpu as pltpu

_N = 2048
_D = 1024
_H = 8
_DH = 128
_BR = 512
_BQ = 2048
_CK = 512
_NB = _N // _BR
_NA = _H * (_N // _BQ)


def _nt(x, w):
    return jax.lax.dot_general(x, w, (((1,), (1,)), ((), ())),
                               preferred_element_type=jnp.float32)


def _mega_body(boxes_ref, feats_ref, w1t_ref, b1_ref, w2_ref, b2_ref,
               wi_ref, bi_ref, wo_ref, bo_ref, wl1_ref, bl1_ref,
               wl2_ref, bl2_ref, out_ref, q_s, k_s, v_s, o_s, acc_ref,
               f_s, fb_ref):
    i = pl.program_id(0)
    bf16 = jnp.bfloat16

    @pl.when(i < _NB)
    def _qkv():
        b = jnp.clip(boxes_ref[...], -10.0, 10.0)
        acc = jnp.broadcast_to(b1_ref[...], (_BR, _D))
        for c in range(4):
            acc = acc + b[:, c:c + 1] * w1t_ref[c:c + 1, :]
        t = jnp.maximum(acc, 0.0)
        pos = _nt(t, w2_ref[...])
        h = feats_ref[...] + pos + b2_ref[...]
        qkv = _nt(h, wi_ref[...]) + bi_ref[...]
        scale = math.log2(math.e) / math.sqrt(_DH)
        q = (qkv[:, :_D] * scale).astype(bf16)
        k = qkv[:, _D:2 * _D].astype(bf16)
        v = qkv[:, 2 * _D:].astype(bf16)
        row = i * _BR
        for hh in range(_H):
            lo, hi = hh * _DH, (hh + 1) * _DH
            q_s[hh, pl.ds(row, _BR), :] = q[:, lo:hi]
            k_s[hh, pl.ds(row, _BR), :] = k[:, lo:hi]
            v_s[hh, pl.ds(row, _BR), :] = v[:, lo:hi]

    @pl.when((i >= _NB) & (i < _NB + _NA))
    def _attn():
        j = i - _NB
        nq = _N // _BQ
        hh = j // nq
        tt = j % nq
        q = q_s[hh, pl.ds(tt * _BQ, _BQ), :]
        o_acc = jnp.zeros((_BQ, _DH), jnp.float32)
        den = jnp.zeros((_BQ, 1), jnp.float32)
        for c in range(_N // _CK):
            kc = k_s[hh, pl.ds(c * _CK, _CK), :]
            vc = v_s[hh, pl.ds(c * _CK, _CK), :]
            s = _nt(q, kc)
            e = jnp.exp2(s.astype(bf16))
            den = den + jnp.sum(e, axis=-1, keepdims=True)
            o_acc = o_acc + jnp.dot(e, vc,
                                    preferred_element_type=jnp.float32)
        o_s[hh, pl.ds(tt * _BQ, _BQ), :] = (o_acc / den).astype(bf16)

    @pl.when(i >= _NB + _NA)
    def _post():
        r = i - (_NB + _NA)
        row = r * _BR

        @pl.when(r == 0)
        def _init():
            acc_ref[...] = jnp.zeros_like(acc_ref)
            f = jax.lax.dot_general(wl1_ref[...], wo_ref[...],
                                    (((1,), (0,)), ((), ())),
                                    preferred_element_type=jnp.float32)
            f_s[...] = f.astype(jnp.bfloat16)
            fb_ref[...] = _nt(bo_ref[...], wl1_ref[...]) + bl1_ref[...]

        o_t = jnp.concatenate(
            [o_s[hh, pl.ds(row, _BR), :] for hh in range(_H)], axis=1)
        h2 = _nt(o_t, f_s[...]) + fb_ref[...]
        h2 = jnp.maximum(h2, 0.0)
        acc_ref[...] += jnp.sum(h2, axis=0, keepdims=True)

        @pl.when(r == _NB - 1)
        def _fin():
            meanv = acc_ref[...] * (1.0 / _N)
            out_ref[...] = _nt(meanv, wl2_ref[...]) + bl2_ref[...]


def kernel(feats, boxes, pos_w1, pos_b1, pos_w2, pos_b2,
           in_proj_w, in_proj_b, out_proj_w, out_proj_b,
           lin1_w, lin1_b, lin2_w, lin2_b):
    f32, bf16 = jnp.float32, jnp.bfloat16
    w1t = pos_w1.T
    b1 = pos_b1.reshape(1, _D)
    b2 = pos_b2.reshape(1, _D)
    bi = in_proj_b.reshape(1, 3 * _D)
    bo = out_proj_b.reshape(1, _D)
    bl1 = lin1_b.reshape(1, _D)
    bl2 = lin2_b.reshape(1, _D)

    def _rows(i):
        return (jnp.minimum(i, _NB - 1), 0)

    def _const(i):
        return (0, 0)

    out = pl.pallas_call(
        _mega_body,
        grid=(_NB + _NA + _NB,),
        in_specs=[
            pl.BlockSpec((_BR, 4), _rows),
            pl.BlockSpec((_BR, _D), _rows),
            pl.BlockSpec((4, _D), _const),
            pl.BlockSpec((1, _D), _const),
            pl.BlockSpec((_D, _D), _const),
            pl.BlockSpec((1, _D), _const),
            pl.BlockSpec((3 * _D, _D), _const),
            pl.BlockSpec((1, 3 * _D), _const),
            pl.BlockSpec((_D, _D), _const),
            pl.BlockSpec((1, _D), _const),
            pl.BlockSpec((_D, _D), _const),
            pl.BlockSpec((1, _D), _const),
            pl.BlockSpec((_D, _D), _const),
            pl.BlockSpec((1, _D), _const),
        ],
        out_specs=pl.BlockSpec((1, _D), _const),
        out_shape=jax.ShapeDtypeStruct((1, _D), f32),
        scratch_shapes=[
            pltpu.VMEM((_H, _N, _DH), bf16),
            pltpu.VMEM((_H, _N, _DH), bf16),
            pltpu.VMEM((_H, _N, _DH), bf16),
            pltpu.VMEM((_H, _N, _DH), bf16),
            pltpu.VMEM((1, _D), f32),
            pltpu.VMEM((_D, _D), bf16),
            pltpu.VMEM((1, _D), f32),
        ],
    )(boxes, feats, w1t, b1, pos_w2, b2, in_proj_w, bi,
      out_proj_w, bo, lin1_w, bl1, lin2_w, bl2)
    return out.reshape(_D)
